# baseline (speedup 1.0000x reference)
import functools

import jax
import jax.numpy as jnp
from jax import lax
from jax.experimental import pallas as pl
from jax.experimental.pallas import tpu as pltpu
from jax.experimental.pallas import tpu_sc as plsc

F = 16
SPC = 16
NC = 2
NW = NC * SPC
CHUNK = 128
CH2 = 2048


def _propagate_h(hs, hsT, sidx, didx, ss, ds, zer, np_, half, nr_ch, ns):
  band = np_ // SPC

  mesh = plsc.VectorSubcoreMesh(core_axis_name="c", subcore_axis_name="s")

  @functools.partial(
      pl.kernel,
      mesh=mesh,
      out_type=[
          jax.ShapeDtypeStruct((F, np_), jnp.float32),
          jax.ShapeDtypeStruct((F, np_), jnp.float32),
          jax.ShapeDtypeStruct((np_, F), jnp.float32),
          jax.ShapeDtypeStruct((np_, F), jnp.float32),
      ],
      scratch_types=[
          pltpu.VMEM((np_,), jnp.float32),
          pltpu.VMEM((np_,), jnp.float32),
          pltpu.VMEM((2 * CH2,), jnp.int32),
          pltpu.VMEM((2 * CH2,), jnp.int32),
          pltpu.VMEM((ns, CHUNK), jnp.int32),
          pltpu.VMEM((ns, CHUNK), jnp.int32),
          pltpu.VMEM((3, CHUNK, F), jnp.float32),
          pltpu.VMEM_SHARED((np_, F), jnp.float32),
          pltpu.SemaphoreType.DMA,
          pltpu.SemaphoreType.DMA,
          pltpu.SemaphoreType.DMA,
      ],
      compiler_params=pltpu.CompilerParams(
          use_tc_tiling_on_sc=False, needs_layout_passes=False),
  )
  def k(hs_hbm, hsT_hbm, sr_hbm, dr_hbm, ss_hbm, ds_hbm, zer_hbm,
        pcaT_hbm, pcbT_hbm, pra_hbm, prb_hbm,
        ht, acc, sv, dv, sv2, dv2, rb, accs, sem, gsem, ssem):
    c = lax.axis_index("c")
    s = lax.axis_index("s")
    rbase = c * half

    t1 = pltpu.async_copy(hsT_hbm.at[s], ht, sem)
    t2 = pltpu.async_copy(ss_hbm.at[c, s], sv2, sem)
    t3 = pltpu.async_copy(ds_hbm.at[c, s], dv2, sem)

    def zbody(i, carry):
      for u in range(8):
        acc[pl.ds((i * 8 + u) * F, F)] = jnp.zeros((F,), jnp.float32)
      return carry

    lax.fori_loop(0, np_ // F // 8, zbody, 0)
    pltpu.sync_copy(zer_hbm, accs.at[pl.ds(s * band, band)])
    t1.wait()
    t2.wait()
    t3.wait()
    plsc.subcore_barrier()

    pltpu.async_copy(sr_hbm.at[pl.ds(rbase, CH2)], sv.at[pl.ds(0, CH2)], sem)
    pltpu.async_copy(dr_hbm.at[pl.ds(rbase, CH2)], dv.at[pl.ds(0, CH2)], sem)
    pltpu.async_copy(hs_hbm.at[sv2.at[0]], rb.at[0], gsem)

    def chunk(j, carry):
      off = lax.rem(j, 2) * CH2
      nxt = CH2 - off
      pltpu.make_async_copy(
          sr_hbm.at[pl.ds(rbase + j * CH2, CH2)],
          sv.at[pl.ds(off, CH2)], sem).wait()
      pltpu.make_async_copy(
          dr_hbm.at[pl.ds(rbase + j * CH2, CH2)],
          dv.at[pl.ds(off, CH2)], sem).wait()

      @pl.when(j + 1 < nr_ch)
      def _():
        pltpu.async_copy(
            sr_hbm.at[pl.ds(rbase + (j + 1) * CH2, CH2)],
            sv.at[pl.ds(nxt, CH2)], sem)
        pltpu.async_copy(
            dr_hbm.at[pl.ds(rbase + (j + 1) * CH2, CH2)],
            dv.at[pl.ds(nxt, CH2)], sem)

      @pl.when(j < ns)
      def _():
        cur = lax.rem(j, 3)

        @pl.when(j >= 2)
        def _():
          pltpu.make_async_copy(
              hs_hbm.at[pl.ds(0, CHUNK)], rb.at[0], ssem).wait()

        pltpu.make_async_copy(
            hs_hbm.at[pl.ds(0, CHUNK)], rb.at[0], gsem).wait()
        pltpu.async_copy(rb.at[cur], accs.at[dv2.at[j]], ssem, add=True)

        @pl.when(j + 1 < ns)
        def _():
          pltpu.async_copy(
              hs_hbm.at[sv2.at[j + 1]], rb.at[lax.rem(j + 1, 3)], gsem)

      @plsc.parallel_loop(0, CH2 // F, step=1, unroll=16)
      def inner(v):
        p = off + v * F
        s16 = sv[pl.ds(p, F)]
        d16 = dv[pl.ds(p, F)]
        vals = plsc.load_gather(ht, [s16])
        plsc.addupdate_scatter(acc, [d16], vals)

      return carry

    lax.fori_loop(0, nr_ch, chunk, 0)

    pltpu.make_async_copy(hs_hbm.at[pl.ds(0, CHUNK)], rb.at[0], ssem).wait()
    pltpu.make_async_copy(hs_hbm.at[pl.ds(0, CHUNK)], rb.at[0], ssem).wait()
    plsc.subcore_barrier()

    @pl.when(c == 0)
    def _():
      pltpu.sync_copy(acc, pcaT_hbm.at[s])
      pltpu.sync_copy(accs.at[pl.ds(s * band, band)],
                      pra_hbm.at[pl.ds(s * band, band)])

    @pl.when(c == 1)
    def _():
      pltpu.sync_copy(acc, pcbT_hbm.at[s])
      pltpu.sync_copy(accs.at[pl.ds(s * band, band)],
                      prb_hbm.at[pl.ds(s * band, band)])

  return k(hs, hsT, sidx, didx, ss, ds, zer)


def _degrees(didx2, np_, ch_w):
  rps = np_ // SPC

  mesh = plsc.VectorSubcoreMesh(core_axis_name="c", subcore_axis_name="s")

  @functools.partial(
      pl.kernel,
      mesh=mesh,
      out_type=[
          jax.ShapeDtypeStruct((np_,), jnp.float32),
          jax.ShapeDtypeStruct((np_,), jnp.float32),
      ],
      scratch_types=[
          pltpu.VMEM_SHARED((np_,), jnp.float32),
          pltpu.VMEM((ch_w, CHUNK), jnp.int32),
          pltpu.VMEM((rps,), jnp.float32),
      ],
      compiler_params=pltpu.CompilerParams(use_tc_tiling_on_sc=False),
  )
  def k(didx_hbm, da_hbm, db_hbm, accd, dv, buf):
    c = lax.axis_index("c")
    s = lax.axis_index("s")
    w = c * SPC + s
    r0 = s * rps
    for i in range(rps // F):
      buf[pl.ds(i * F, F)] = jnp.full((F,), 1.0, jnp.float32)
    pltpu.sync_copy(buf, accd.at[pl.ds(r0, rps)])
    pltpu.sync_copy(didx_hbm.at[pl.ds(w * ch_w, ch_w)], dv)
    plsc.subcore_barrier()

    def body(j, carry):
      pltpu.sync_copy(buf.at[pl.ds(0, CHUNK)], accd.at[dv.at[j]], add=True)
      return carry

    lax.fori_loop(0, ch_w, body, 0)

    plsc.subcore_barrier()

    @pl.when(c == 0)
    def _():
      pltpu.sync_copy(accd.at[pl.ds(r0, rps)], da_hbm.at[pl.ds(r0, rps)])

    @pl.when(c == 1)
    def _():
      pltpu.sync_copy(accd.at[pl.ds(r0, rps)], db_hbm.at[pl.ds(r0, rps)])

  return k(didx2)


def _tc_prep(da, db, x, w1, np_):
  def body(da_ref, db_ref, x_ref, w1_ref, dinv_ref, hsT_ref, hs_ref):
    deg = da_ref[...] + db_ref[...] - 1.0
    dinv = lax.rsqrt(deg)
    hT = lax.dot_general(
        w1_ref[...], x_ref[...],
        dimension_numbers=(((0,), (1,)), ((), ())),
        preferred_element_type=jnp.float32)
    hsT = dinv * hT
    dinv_ref[...] = dinv
    hsT_ref[...] = hsT
    hs_ref[...] = hsT.T

  return pl.pallas_call(
      body,
      out_shape=[
          jax.ShapeDtypeStruct((1, np_), jnp.float32),
          jax.ShapeDtypeStruct((F, np_), jnp.float32),
          jax.ShapeDtypeStruct((np_, F), jnp.float32),
      ],
  )(da, db, x, w1)


def _tc_layer(dinv, pcaT, pcbT, pra, prb, hsT, b1c, w2pT, np_):
  def body(dinv_ref, pa_ref, pb_ref, ra_ref, rb_ref, hsp_ref, b1_ref, w2t_ref,
           out_ref, outr_ref):
    tot = (pa_ref[...] + pb_ref[...] + hsp_ref[...]
           + (ra_ref[...] + rb_ref[...]).T)
    t = dinv_ref[...] * tot + b1_ref[...]
    h2 = jnp.maximum(t, 0.0)
    hs2T = dinv_ref[...] * jnp.dot(
        w2t_ref[...], h2, preferred_element_type=jnp.float32)
    out_ref[...] = hs2T
    outr_ref[...] = hs2T.T

  return pl.pallas_call(
      body,
      out_shape=[
          jax.ShapeDtypeStruct((F, np_), jnp.float32),
          jax.ShapeDtypeStruct((np_, F), jnp.float32),
      ],
  )(dinv, pcaT, pcbT, pra, prb, hsT, b1c, w2pT)


def _tc_final(dinv, qaT, qbT, qra, qrb, hs2T, b2c, np_, c_):
  def body(dinv_ref, qa_ref, qb_ref, ra_ref, rb_ref, hsp_ref, b2_ref, out_ref):
    tot = (qa_ref[...] + qb_ref[...] + hsp_ref[...]
           + (ra_ref[...] + rb_ref[...]).T)
    t = dinv_ref[...] * tot + b2_ref[...]
    r = jnp.maximum(t, 0.0)
    row = lax.broadcasted_iota(jnp.int32, (F, np_), 0)
    valid = row < c_
    rm = jnp.where(valid, r, jnp.float32(-1e30))
    m = jnp.max(rm, axis=0, keepdims=True)
    e = jnp.where(valid, jnp.exp(rm - m), 0.0)
    ssum = jnp.sum(e, axis=0, keepdims=True)
    res = rm - m - jnp.log(ssum)
    out_ref[...] = res.T

  return pl.pallas_call(
      body,
      out_shape=jax.ShapeDtypeStruct((np_, F), jnp.float32),
  )(dinv, qaT, qbT, qra, qrb, hs2T, b2c)


def kernel(x, edge_index, W1, b1, W2, b2):
  n, d = x.shape
  h = W1.shape[1]
  c_ = W2.shape[1]
  assert h == F
  e = edge_index.shape[1]

  np_ = ((n + 255) // 256) * 256
  nch = -(-e // (NC * CH2))
  ep = NC * nch * CH2
  ch_w = ep // (NW * CHUNK)

  src = edge_index[0]
  dst = edge_index[1]
  pad = jnp.full((ep - e,), n, dtype=jnp.int32)
  sidx = jnp.concatenate([src, pad])
  didx = jnp.concatenate([dst, pad])

  xp = jnp.pad(x, ((0, np_ - n), (0, 0)))

  half = nch * CH2
  ns = (nch * 46) // 100
  nr_ch = nch - ns
  er = nr_ch * CH2
  ss = sidx.reshape(NC, half)[:, er:].reshape(NC, SPC, ns, CHUNK)
  ds = didx.reshape(NC, half)[:, er:].reshape(NC, SPC, ns, CHUNK)
  zer = jnp.zeros((np_ // SPC, F), jnp.float32)

  da, db = _degrees(didx.reshape(NW * ch_w, CHUNK), np_, ch_w)
  dinv, hsT, hs = _tc_prep(da.reshape(1, np_), db.reshape(1, np_), xp, W1, np_)
  paT, pbT, pra, prb = _propagate_h(hs, hsT, sidx, didx, ss, ds, zer,
                                    np_, half, nr_ch, ns)
  w2pT = jnp.pad(W2, ((0, 0), (0, F - c_))).T
  hs2T, hs2 = _tc_layer(dinv, paT, pbT, pra, prb, hsT, b1.reshape(F, 1),
                        w2pT, np_)
  qaT, qbT, qra, qrb = _propagate_h(hs2, hs2T, sidx, didx, ss, ds, zer,
                                    np_, half, nr_ch, ns)
  b2c = jnp.pad(b2, (0, F - c_)).reshape(F, 1)
  out = _tc_final(dinv, qaT, qbT, qra, qrb, hs2T, b2c, np_, c_)
  return out[:n, :c_]


# speedup vs baseline: 1.0441x; 1.0441x over previous
import functools

import jax
import jax.numpy as jnp
from jax import lax
from jax.experimental import pallas as pl
from jax.experimental.pallas import tpu as pltpu
from jax.experimental.pallas import tpu_sc as plsc

F = 16
SPC = 16
NC = 2
NW = NC * SPC
CHUNK = 128
CH2 = 2048


def _propagate_h(hs, hsT, sr, dr, ss, ds, zer, np_, er, nr_ch, ns):
  band = np_ // SPC

  mesh = plsc.VectorSubcoreMesh(core_axis_name="c", subcore_axis_name="s")

  @functools.partial(
      pl.kernel,
      mesh=mesh,
      out_type=[
          jax.ShapeDtypeStruct((F, np_), jnp.float32),
          jax.ShapeDtypeStruct((F, np_), jnp.float32),
          jax.ShapeDtypeStruct((np_, F), jnp.float32),
          jax.ShapeDtypeStruct((np_, F), jnp.float32),
      ],
      scratch_types=[
          pltpu.VMEM((np_,), jnp.float32),
          pltpu.VMEM((np_,), jnp.float32),
          pltpu.VMEM((2 * CH2,), jnp.int32),
          pltpu.VMEM((2 * CH2,), jnp.int32),
          pltpu.VMEM((ns, CHUNK), jnp.int32),
          pltpu.VMEM((ns, CHUNK), jnp.int32),
          pltpu.VMEM((3, CHUNK, F), jnp.float32),
          pltpu.VMEM_SHARED((np_, F), jnp.float32),
          pltpu.SemaphoreType.DMA,
          pltpu.SemaphoreType.DMA,
          pltpu.SemaphoreType.DMA,
      ],
      compiler_params=pltpu.CompilerParams(
          use_tc_tiling_on_sc=False, needs_layout_passes=False),
  )
  def k(hs_hbm, hsT_hbm, sr_hbm, dr_hbm, ss_hbm, ds_hbm, zer_hbm,
        pcaT_hbm, pcbT_hbm, pra_hbm, prb_hbm,
        ht, acc, sv, dv, sv2, dv2, rb, accs, sem, gsem, ssem):
    c = lax.axis_index("c")
    s = lax.axis_index("s")

    t1 = pltpu.async_copy(hsT_hbm.at[s], ht, sem)
    t2 = pltpu.async_copy(ss_hbm.at[c, s], sv2, sem)
    t3 = pltpu.async_copy(ds_hbm.at[c, s], dv2, sem)

    def zbody(i, carry):
      for u in range(8):
        acc[pl.ds((i * 8 + u) * F, F)] = jnp.zeros((F,), jnp.float32)
      return carry

    lax.fori_loop(0, np_ // F // 8, zbody, 0)
    pltpu.sync_copy(zer_hbm, accs.at[pl.ds(s * band, band)])
    t1.wait()
    t2.wait()
    t3.wait()
    plsc.subcore_barrier()

    srj = sr_hbm.at[c]
    drj = dr_hbm.at[c]
    pltpu.async_copy(srj.at[pl.ds(0, CH2)], sv.at[pl.ds(0, CH2)], sem)
    pltpu.async_copy(drj.at[pl.ds(0, CH2)], dv.at[pl.ds(0, CH2)], sem)
    pltpu.async_copy(hs_hbm.at[sv2.at[0]], rb.at[0], gsem)

    def chunk(j, carry):
      off = lax.rem(j, 2) * CH2
      nxt = CH2 - off
      pltpu.make_async_copy(
          srj.at[pl.ds(j * CH2, CH2)], sv.at[pl.ds(off, CH2)], sem).wait()
      pltpu.make_async_copy(
          drj.at[pl.ds(j * CH2, CH2)], dv.at[pl.ds(off, CH2)], sem).wait()

      @pl.when(j + 1 < nr_ch)
      def _():
        pltpu.async_copy(
            srj.at[pl.ds((j + 1) * CH2, CH2)], sv.at[pl.ds(nxt, CH2)], sem)
        pltpu.async_copy(
            drj.at[pl.ds((j + 1) * CH2, CH2)], dv.at[pl.ds(nxt, CH2)], sem)

      @pl.when(j < ns)
      def _():
        cur = lax.rem(j, 3)

        @pl.when(j >= 2)
        def _():
          pltpu.make_async_copy(
              hs_hbm.at[pl.ds(0, CHUNK)], rb.at[0], ssem).wait()

        pltpu.make_async_copy(
            hs_hbm.at[pl.ds(0, CHUNK)], rb.at[0], gsem).wait()
        pltpu.async_copy(rb.at[cur], accs.at[dv2.at[j]], ssem, add=True)

        @pl.when(j + 1 < ns)
        def _():
          pltpu.async_copy(
              hs_hbm.at[sv2.at[j + 1]], rb.at[lax.rem(j + 1, 3)], gsem)

      @plsc.parallel_loop(0, CH2 // F, step=1, unroll=16)
      def inner(v):
        p = off + v * F
        s16 = sv[pl.ds(p, F)]
        d16 = dv[pl.ds(p, F)]
        vals = plsc.load_gather(ht, [s16])
        plsc.addupdate_scatter(acc, [d16], vals)

      return carry

    lax.fori_loop(0, nr_ch, chunk, 0)

    pltpu.make_async_copy(hs_hbm.at[pl.ds(0, CHUNK)], rb.at[0], ssem).wait()
    pltpu.make_async_copy(hs_hbm.at[pl.ds(0, CHUNK)], rb.at[0], ssem).wait()
    plsc.subcore_barrier()

    @pl.when(c == 0)
    def _():
      pltpu.sync_copy(acc, pcaT_hbm.at[s])
      pltpu.sync_copy(accs.at[pl.ds(s * band, band)],
                      pra_hbm.at[pl.ds(s * band, band)])

    @pl.when(c == 1)
    def _():
      pltpu.sync_copy(acc, pcbT_hbm.at[s])
      pltpu.sync_copy(accs.at[pl.ds(s * band, band)],
                      prb_hbm.at[pl.ds(s * band, band)])

  return k(hs, hsT, sr, dr, ss, ds, zer)


def _degrees(didx2, np_, ch_w):
  rps = np_ // SPC

  mesh = plsc.VectorSubcoreMesh(core_axis_name="c", subcore_axis_name="s")

  @functools.partial(
      pl.kernel,
      mesh=mesh,
      out_type=[
          jax.ShapeDtypeStruct((np_,), jnp.float32),
          jax.ShapeDtypeStruct((np_,), jnp.float32),
      ],
      scratch_types=[
          pltpu.VMEM_SHARED((np_,), jnp.float32),
          pltpu.VMEM((ch_w, CHUNK), jnp.int32),
          pltpu.VMEM((rps,), jnp.float32),
      ],
      compiler_params=pltpu.CompilerParams(use_tc_tiling_on_sc=False),
  )
  def k(didx_hbm, da_hbm, db_hbm, accd, dv, buf):
    c = lax.axis_index("c")
    s = lax.axis_index("s")
    w = c * SPC + s
    r0 = s * rps
    for i in range(rps // F):
      buf[pl.ds(i * F, F)] = jnp.full((F,), 1.0, jnp.float32)
    pltpu.sync_copy(buf, accd.at[pl.ds(r0, rps)])
    pltpu.sync_copy(didx_hbm.at[pl.ds(w * ch_w, ch_w)], dv)
    plsc.subcore_barrier()

    def body(j, carry):
      pltpu.sync_copy(buf.at[pl.ds(0, CHUNK)], accd.at[dv.at[j]], add=True)
      return carry

    lax.fori_loop(0, ch_w, body, 0)

    plsc.subcore_barrier()

    @pl.when(c == 0)
    def _():
      pltpu.sync_copy(accd.at[pl.ds(r0, rps)], da_hbm.at[pl.ds(r0, rps)])

    @pl.when(c == 1)
    def _():
      pltpu.sync_copy(accd.at[pl.ds(r0, rps)], db_hbm.at[pl.ds(r0, rps)])

  return k(didx2)


def _tc_prep(da, db, x, w1, np_):
  def body(da_ref, db_ref, x_ref, w1_ref, dinv_ref, hsT_ref, hs_ref):
    deg = da_ref[...] + db_ref[...] - 1.0
    dinv = lax.rsqrt(deg)
    hT = lax.dot_general(
        w1_ref[...], x_ref[...],
        dimension_numbers=(((0,), (1,)), ((), ())),
        preferred_element_type=jnp.float32)
    hsT = dinv * hT
    dinv_ref[...] = dinv
    hsT_ref[...] = hsT
    hs_ref[...] = hsT.T

  return pl.pallas_call(
      body,
      out_shape=[
          jax.ShapeDtypeStruct((1, np_), jnp.float32),
          jax.ShapeDtypeStruct((F, np_), jnp.float32),
          jax.ShapeDtypeStruct((np_, F), jnp.float32),
      ],
  )(da, db, x, w1)


def _tc_layer(dinv, pcaT, pcbT, pra, prb, hsT, b1c, w2pT, np_):
  def body(dinv_ref, pa_ref, pb_ref, ra_ref, rb_ref, hsp_ref, b1_ref, w2t_ref,
           out_ref, outr_ref):
    tot = (pa_ref[...] + pb_ref[...] + hsp_ref[...]
           + (ra_ref[...] + rb_ref[...]).T)
    t = dinv_ref[...] * tot + b1_ref[...]
    h2 = jnp.maximum(t, 0.0)
    hs2T = dinv_ref[...] * jnp.dot(
        w2t_ref[...], h2, preferred_element_type=jnp.float32)
    out_ref[...] = hs2T
    outr_ref[...] = hs2T.T

  return pl.pallas_call(
      body,
      out_shape=[
          jax.ShapeDtypeStruct((F, np_), jnp.float32),
          jax.ShapeDtypeStruct((np_, F), jnp.float32),
      ],
  )(dinv, pcaT, pcbT, pra, prb, hsT, b1c, w2pT)


def _tc_final(dinv, qaT, qbT, qra, qrb, hs2T, b2c, np_, c_):
  def body(dinv_ref, qa_ref, qb_ref, ra_ref, rb_ref, hsp_ref, b2_ref, out_ref):
    tot = (qa_ref[...] + qb_ref[...] + hsp_ref[...]
           + (ra_ref[...] + rb_ref[...]).T)
    t = dinv_ref[...] * tot + b2_ref[...]
    r = jnp.maximum(t, 0.0)
    row = lax.broadcasted_iota(jnp.int32, (F, np_), 0)
    valid = row < c_
    rm = jnp.where(valid, r, jnp.float32(-1e30))
    m = jnp.max(rm, axis=0, keepdims=True)
    e = jnp.where(valid, jnp.exp(rm - m), 0.0)
    ssum = jnp.sum(e, axis=0, keepdims=True)
    res = rm - m - jnp.log(ssum)
    out_ref[...] = res.T

  return pl.pallas_call(
      body,
      out_shape=jax.ShapeDtypeStruct((np_, F), jnp.float32),
  )(dinv, qaT, qbT, qra, qrb, hs2T, b2c)


def kernel(x, edge_index, W1, b1, W2, b2):
  n, d = x.shape
  h = W1.shape[1]
  c_ = W2.shape[1]
  assert h == F
  e = edge_index.shape[1]

  np_ = ((n + 255) // 256) * 256
  nch = -(-e // (NC * CH2))
  ep = NC * nch * CH2
  ch_w = ep // (NW * CHUNK)

  src = edge_index[0]
  dst = edge_index[1]
  pad = jnp.full((ep - e,), n, dtype=jnp.int32)
  sidx = jnp.concatenate([src, pad])
  didx = jnp.concatenate([dst, pad])

  xp = jnp.pad(x, ((0, np_ - n), (0, 0)))

  half = nch * CH2
  ns = (nch * 46) // 100
  nr_ch = nch - ns
  er = nr_ch * CH2
  sh = sidx.reshape(NC, half)
  dh = didx.reshape(NC, half)
  sr = sh[:, :er]
  dr = dh[:, :er]
  ss = sh[:, er:].reshape(NC, SPC, ns, CHUNK)
  ds = dh[:, er:].reshape(NC, SPC, ns, CHUNK)
  zer = jnp.zeros((np_ // SPC, F), jnp.float32)

  da, db = _degrees(didx.reshape(NW * ch_w, CHUNK), np_, ch_w)
  dinv, hsT, hs = _tc_prep(da.reshape(1, np_), db.reshape(1, np_), xp, W1, np_)
  paT, pbT, pra, prb = _propagate_h(hs, hsT, sr, dr, ss, ds, zer,
                                    np_, er, nr_ch, ns)
  w2pT = jnp.pad(W2, ((0, 0), (0, F - c_))).T
  hs2T, hs2 = _tc_layer(dinv, paT, pbT, pra, prb, hsT, b1.reshape(F, 1),
                        w2pT, np_)
  qaT, qbT, qra, qrb = _propagate_h(hs2, hs2T, sr, dr, ss, ds, zer,
                                    np_, er, nr_ch, ns)
  b2c = jnp.pad(b2, (0, F - c_)).reshape(F, 1)
  out = _tc_final(dinv, qaT, qbT, qra, qrb, hs2T, b2c, np_, c_)
  return out[:n, :c_]


# speedup vs baseline: 1.0460x; 1.0018x over previous
import functools

import jax
import jax.numpy as jnp
from jax import lax
from jax.experimental import pallas as pl
from jax.experimental.pallas import tpu as pltpu
from jax.experimental.pallas import tpu_sc as plsc

F = 16
SPC = 16
NC = 2
NW = NC * SPC
CHUNK = 128
CH2 = 2048


def _propagate_h(hs, hsT, sr, dr, ss, ds, zer, np_, er, nr_ch, ns):
  band = np_ // SPC

  mesh = plsc.VectorSubcoreMesh(core_axis_name="c", subcore_axis_name="s")

  @functools.partial(
      pl.kernel,
      mesh=mesh,
      out_type=[
          jax.ShapeDtypeStruct((F, np_), jnp.float32),
          jax.ShapeDtypeStruct((F, np_), jnp.float32),
          jax.ShapeDtypeStruct((np_, F), jnp.float32),
          jax.ShapeDtypeStruct((np_, F), jnp.float32),
      ],
      scratch_types=[
          pltpu.VMEM((np_,), jnp.float32),
          pltpu.VMEM((np_,), jnp.float32),
          pltpu.VMEM((2 * CH2,), jnp.int32),
          pltpu.VMEM((2 * CH2,), jnp.int32),
          pltpu.VMEM((ns, CHUNK), jnp.int32),
          pltpu.VMEM((ns, CHUNK), jnp.int32),
          pltpu.VMEM((3, CHUNK, F), jnp.float32),
          pltpu.VMEM_SHARED((np_, F), jnp.float32),
          pltpu.SemaphoreType.DMA,
          pltpu.SemaphoreType.DMA,
          pltpu.SemaphoreType.DMA,
      ],
      compiler_params=pltpu.CompilerParams(
          use_tc_tiling_on_sc=False, needs_layout_passes=False),
  )
  def k(hs_hbm, hsT_hbm, sr_hbm, dr_hbm, ss_hbm, ds_hbm, zer_hbm,
        pcaT_hbm, pcbT_hbm, pra_hbm, prb_hbm,
        ht, acc, sv, dv, sv2, dv2, rb, accs, sem, gsem, ssem):
    c = lax.axis_index("c")
    s = lax.axis_index("s")

    t1 = pltpu.async_copy(hsT_hbm.at[s], ht, sem)
    t2 = pltpu.async_copy(ss_hbm.at[c, s], sv2, sem)
    t3 = pltpu.async_copy(ds_hbm.at[c, s], dv2, sem)

    def zbody(i, carry):
      for u in range(8):
        acc[pl.ds((i * 8 + u) * F, F)] = jnp.zeros((F,), jnp.float32)
      return carry

    lax.fori_loop(0, np_ // F // 8, zbody, 0)
    pltpu.sync_copy(zer_hbm, accs.at[pl.ds(s * band, band)])
    t1.wait()
    t2.wait()
    t3.wait()
    plsc.subcore_barrier()

    srj = sr_hbm.at[c]
    drj = dr_hbm.at[c]
    pltpu.async_copy(srj.at[pl.ds(0, CH2)], sv.at[pl.ds(0, CH2)], sem)
    pltpu.async_copy(drj.at[pl.ds(0, CH2)], dv.at[pl.ds(0, CH2)], sem)
    pltpu.async_copy(hs_hbm.at[sv2.at[0]], rb.at[0], gsem)

    def chunk(j, carry):
      off = lax.rem(j, 2) * CH2
      nxt = CH2 - off
      pltpu.make_async_copy(
          srj.at[pl.ds(j * CH2, CH2)], sv.at[pl.ds(off, CH2)], sem).wait()
      pltpu.make_async_copy(
          drj.at[pl.ds(j * CH2, CH2)], dv.at[pl.ds(off, CH2)], sem).wait()

      @pl.when(j + 1 < nr_ch)
      def _():
        pltpu.async_copy(
            srj.at[pl.ds((j + 1) * CH2, CH2)], sv.at[pl.ds(nxt, CH2)], sem)
        pltpu.async_copy(
            drj.at[pl.ds((j + 1) * CH2, CH2)], dv.at[pl.ds(nxt, CH2)], sem)

      @pl.when(j < ns)
      def _():
        cur = lax.rem(j, 3)

        @pl.when(j >= 2)
        def _():
          pltpu.make_async_copy(
              hs_hbm.at[pl.ds(0, CHUNK)], rb.at[0], ssem).wait()

        pltpu.make_async_copy(
            hs_hbm.at[pl.ds(0, CHUNK)], rb.at[0], gsem).wait()
        pltpu.async_copy(rb.at[cur], accs.at[dv2.at[j]], ssem, add=True)

        @pl.when(j + 1 < ns)
        def _():
          pltpu.async_copy(
              hs_hbm.at[sv2.at[j + 1]], rb.at[lax.rem(j + 1, 3)], gsem)

      @plsc.parallel_loop(0, CH2 // F, step=1, unroll=16)
      def inner(v):
        p = off + v * F
        s16 = sv[pl.ds(p, F)]
        d16 = dv[pl.ds(p, F)]
        vals = plsc.load_gather(ht, [s16])
        plsc.addupdate_scatter(acc, [d16], vals)

      return carry

    lax.fori_loop(0, nr_ch, chunk, 0)

    pltpu.make_async_copy(hs_hbm.at[pl.ds(0, CHUNK)], rb.at[0], ssem).wait()
    pltpu.make_async_copy(hs_hbm.at[pl.ds(0, CHUNK)], rb.at[0], ssem).wait()
    plsc.subcore_barrier()

    @pl.when(c == 0)
    def _():
      pltpu.sync_copy(acc, pcaT_hbm.at[s])
      pltpu.sync_copy(accs.at[pl.ds(s * band, band)],
                      pra_hbm.at[pl.ds(s * band, band)])

    @pl.when(c == 1)
    def _():
      pltpu.sync_copy(acc, pcbT_hbm.at[s])
      pltpu.sync_copy(accs.at[pl.ds(s * band, band)],
                      prb_hbm.at[pl.ds(s * band, band)])

  return k(hs, hsT, sr, dr, ss, ds, zer)


def _degrees(didx2, np_, ch_w):
  rps = np_ // SPC

  mesh = plsc.VectorSubcoreMesh(core_axis_name="c", subcore_axis_name="s")

  @functools.partial(
      pl.kernel,
      mesh=mesh,
      out_type=[
          jax.ShapeDtypeStruct((np_,), jnp.float32),
          jax.ShapeDtypeStruct((np_,), jnp.float32),
      ],
      scratch_types=[
          pltpu.VMEM_SHARED((np_,), jnp.float32),
          pltpu.VMEM((ch_w, CHUNK), jnp.int32),
          pltpu.VMEM((rps,), jnp.float32),
          pltpu.SemaphoreType.DMA,
      ],
      compiler_params=pltpu.CompilerParams(use_tc_tiling_on_sc=False),
  )
  def k(didx_hbm, da_hbm, db_hbm, accd, dv, buf, sem):
    c = lax.axis_index("c")
    s = lax.axis_index("s")
    w = c * SPC + s
    r0 = s * rps
    for i in range(rps // F):
      buf[pl.ds(i * F, F)] = jnp.full((F,), 1.0, jnp.float32)
    pltpu.sync_copy(buf, accd.at[pl.ds(r0, rps)])
    pltpu.sync_copy(didx_hbm.at[pl.ds(w * ch_w, ch_w)], dv)
    plsc.subcore_barrier()

    def body(j, carry):
      pltpu.async_copy(buf.at[pl.ds(0, CHUNK)], accd.at[dv.at[j]], sem,
                       add=True)
      return carry

    lax.fori_loop(0, ch_w, body, 0)

    def drain(j, carry):
      pltpu.make_async_copy(
          didx_hbm.at[pl.ds(0, CHUNK)], dv.at[0], sem).wait()
      return carry

    lax.fori_loop(0, ch_w, drain, 0)

    plsc.subcore_barrier()

    @pl.when(c == 0)
    def _():
      pltpu.sync_copy(accd.at[pl.ds(r0, rps)], da_hbm.at[pl.ds(r0, rps)])

    @pl.when(c == 1)
    def _():
      pltpu.sync_copy(accd.at[pl.ds(r0, rps)], db_hbm.at[pl.ds(r0, rps)])

  return k(didx2)


def _tc_prep(da, db, x, w1, np_):
  def body(da_ref, db_ref, x_ref, w1_ref, dinv_ref, hsT_ref, hs_ref):
    deg = da_ref[...] + db_ref[...] - 1.0
    dinv = lax.rsqrt(deg)
    hT = lax.dot_general(
        w1_ref[...], x_ref[...],
        dimension_numbers=(((0,), (1,)), ((), ())),
        preferred_element_type=jnp.float32)
    hsT = dinv * hT
    dinv_ref[...] = dinv
    hsT_ref[...] = hsT
    hs_ref[...] = hsT.T

  return pl.pallas_call(
      body,
      out_shape=[
          jax.ShapeDtypeStruct((1, np_), jnp.float32),
          jax.ShapeDtypeStruct((F, np_), jnp.float32),
          jax.ShapeDtypeStruct((np_, F), jnp.float32),
      ],
  )(da, db, x, w1)


def _tc_layer(dinv, pcaT, pcbT, pra, prb, hsT, b1c, w2pT, np_):
  def body(dinv_ref, pa_ref, pb_ref, ra_ref, rb_ref, hsp_ref, b1_ref, w2t_ref,
           out_ref, outr_ref):
    tot = (pa_ref[...] + pb_ref[...] + hsp_ref[...]
           + (ra_ref[...] + rb_ref[...]).T)
    t = dinv_ref[...] * tot + b1_ref[...]
    h2 = jnp.maximum(t, 0.0)
    hs2T = dinv_ref[...] * jnp.dot(
        w2t_ref[...], h2, preferred_element_type=jnp.float32)
    out_ref[...] = hs2T
    outr_ref[...] = hs2T.T

  return pl.pallas_call(
      body,
      out_shape=[
          jax.ShapeDtypeStruct((F, np_), jnp.float32),
          jax.ShapeDtypeStruct((np_, F), jnp.float32),
      ],
  )(dinv, pcaT, pcbT, pra, prb, hsT, b1c, w2pT)


def _tc_final(dinv, qaT, qbT, qra, qrb, hs2T, b2c, np_, c_):
  def body(dinv_ref, qa_ref, qb_ref, ra_ref, rb_ref, hsp_ref, b2_ref, out_ref):
    tot = (qa_ref[...] + qb_ref[...] + hsp_ref[...]
           + (ra_ref[...] + rb_ref[...]).T)
    t = dinv_ref[...] * tot + b2_ref[...]
    r = jnp.maximum(t, 0.0)
    row = lax.broadcasted_iota(jnp.int32, (F, np_), 0)
    valid = row < c_
    rm = jnp.where(valid, r, jnp.float32(-1e30))
    m = jnp.max(rm, axis=0, keepdims=True)
    e = jnp.where(valid, jnp.exp(rm - m), 0.0)
    ssum = jnp.sum(e, axis=0, keepdims=True)
    res = rm - m - jnp.log(ssum)
    out_ref[...] = res.T

  return pl.pallas_call(
      body,
      out_shape=jax.ShapeDtypeStruct((np_, F), jnp.float32),
  )(dinv, qaT, qbT, qra, qrb, hs2T, b2c)


def kernel(x, edge_index, W1, b1, W2, b2):
  n, d = x.shape
  h = W1.shape[1]
  c_ = W2.shape[1]
  assert h == F
  e = edge_index.shape[1]

  np_ = ((n + 255) // 256) * 256
  nch = -(-e // (NC * CH2))
  ep = NC * nch * CH2
  ch_w = ep // (NW * CHUNK)

  src = edge_index[0]
  dst = edge_index[1]
  pad = jnp.full((ep - e,), n, dtype=jnp.int32)
  sidx = jnp.concatenate([src, pad])
  didx = jnp.concatenate([dst, pad])

  xp = jnp.pad(x, ((0, np_ - n), (0, 0)))

  half = nch * CH2
  ns = (nch * 46) // 100
  nr_ch = nch - ns
  er = nr_ch * CH2
  sh = sidx.reshape(NC, half)
  dh = didx.reshape(NC, half)
  sr = sh[:, :er]
  dr = dh[:, :er]
  ss = sh[:, er:].reshape(NC, SPC, ns, CHUNK)
  ds = dh[:, er:].reshape(NC, SPC, ns, CHUNK)
  zer = jnp.zeros((np_ // SPC, F), jnp.float32)

  da, db = _degrees(didx.reshape(NW * ch_w, CHUNK), np_, ch_w)
  dinv, hsT, hs = _tc_prep(da.reshape(1, np_), db.reshape(1, np_), xp, W1, np_)
  paT, pbT, pra, prb = _propagate_h(hs, hsT, sr, dr, ss, ds, zer,
                                    np_, er, nr_ch, ns)
  w2pT = jnp.pad(W2, ((0, 0), (0, F - c_))).T
  hs2T, hs2 = _tc_layer(dinv, paT, pbT, pra, prb, hsT, b1.reshape(F, 1),
                        w2pT, np_)
  qaT, qbT, qra, qrb = _propagate_h(hs2, hs2T, sr, dr, ss, ds, zer,
                                    np_, er, nr_ch, ns)
  b2c = jnp.pad(b2, (0, F - c_)).reshape(F, 1)
  out = _tc_final(dinv, qaT, qbT, qra, qrb, hs2T, b2c, np_, c_)
  return out[:n, :c_]
